# baseline (speedup 1.0000x reference)
import jax
import jax.numpy as jnp
from jax.experimental import pallas as pl
from jax.experimental.pallas import tpu as pltpu

_NUM_STUDENTS = 4096
_NUM_QUESTIONS = 16384
_NUM_CONCEPTS = 128
_QB = 4096
_SB = 1024
_DC = 512


def _fwd(a_ref, d_ref, c_ref, o_ref, oh_ref):
    @pl.when(pl.program_id(1) == 0)
    def _build_onehot():
        c = c_ref[0]
        k2 = jax.lax.broadcasted_iota(jnp.int32, (2 * _NUM_CONCEPTS, _QB), 0)
        oh_ref[...] = (c == k2).astype(jnp.bfloat16)
        qi = jax.lax.broadcasted_iota(jnp.int32, (_DC, _DC), 0)
        qj = jax.lax.broadcasted_iota(jnp.int32, (_DC, _DC), 1)
        eye = qi == qj
        for j in range(_QB // _DC):
            ohj = oh_ref[: _NUM_CONCEPTS, pl.ds(j * _DC, _DC)]
            m = jnp.dot(d_ref[pl.ds(j * _DC, _DC), :], ohj,
                        preferred_element_type=jnp.float32)
            ndh = -jnp.sum(jnp.where(eye, m, 0.0), axis=0, keepdims=True)
            oh_ref[_NUM_CONCEPTS : _NUM_CONCEPTS + 1, pl.ds(j * _DC, _DC)] = (
                ndh.astype(jnp.bfloat16))

    z = jnp.dot(a_ref[...], oh_ref[...], preferred_element_type=jnp.float32)
    o_ref[...] = 0.6 + 0.4 * jnp.tanh(z)


def kernel(x, A, D, concepts_q, concepts_c):
    nq = _NUM_QUESTIONS // _QB
    ns = _NUM_STUDENTS // _SB
    c3 = concepts_c.reshape(nq, 1, _QB)
    a_aug = jnp.zeros((_NUM_STUDENTS, 2 * _NUM_CONCEPTS), jnp.bfloat16)
    a_aug = a_aug.at[:, :_NUM_CONCEPTS].set((0.5 * A).astype(jnp.bfloat16))
    a_aug = a_aug.at[:, _NUM_CONCEPTS].set(jnp.bfloat16(1.0))
    d_half = (0.5 * D).astype(jnp.bfloat16)
    return pl.pallas_call(
        _fwd,
        grid=(nq, ns),
        in_specs=[
            pl.BlockSpec((_SB, 2 * _NUM_CONCEPTS), lambda q, s: (s, 0)),
            pl.BlockSpec((_QB, _NUM_CONCEPTS), lambda q, s: (q, 0)),
            pl.BlockSpec((1, 1, _QB), lambda q, s: (q, 0, 0)),
        ],
        out_specs=pl.BlockSpec((_SB, _QB), lambda q, s: (s, q)),
        out_shape=jax.ShapeDtypeStruct((_NUM_STUDENTS, _NUM_QUESTIONS),
                                       jnp.float32),
        scratch_shapes=[pltpu.VMEM((2 * _NUM_CONCEPTS, _QB), jnp.bfloat16)],
    )(a_aug, d_half, c3)


# speedup vs baseline: 1.0318x; 1.0318x over previous
import jax
import jax.numpy as jnp
from jax.experimental import pallas as pl
from jax.experimental.pallas import tpu as pltpu

_NUM_STUDENTS = 4096
_NUM_QUESTIONS = 16384
_NUM_CONCEPTS = 128
_QB = 1024


def _fwd(a_ref, d_ref, c_ref, o_ref):
    c = c_ref[0]
    oh = (c == jax.lax.broadcasted_iota(jnp.int32, (_NUM_CONCEPTS, _QB), 0))
    oh = oh.astype(jnp.bfloat16)
    m = jnp.dot(d_ref[...], oh, preferred_element_type=jnp.float32)
    qi = jax.lax.broadcasted_iota(jnp.int32, (_QB, _QB), 0)
    qj = jax.lax.broadcasted_iota(jnp.int32, (_QB, _QB), 1)
    neg_dh = -jnp.sum(jnp.where(qi == qj, m, 0.0), axis=0, keepdims=True)
    neg_dh = neg_dh.astype(jnp.bfloat16)
    pad = jnp.zeros((_NUM_CONCEPTS - 1, _QB), jnp.bfloat16)
    oh_aug = jnp.concatenate([oh, neg_dh, pad], axis=0)
    z = jnp.dot(a_ref[...], oh_aug, preferred_element_type=jnp.float32)
    o_ref[...] = 0.6 + 0.4 * jnp.tanh(z)


def kernel(x, A, D, concepts_q, concepts_c):
    nb = _NUM_QUESTIONS // _QB
    c3 = concepts_c.reshape(nb, 1, _QB)
    a_aug = jnp.zeros((_NUM_STUDENTS, 2 * _NUM_CONCEPTS), jnp.bfloat16)
    a_aug = a_aug.at[:, :_NUM_CONCEPTS].set((0.5 * A).astype(jnp.bfloat16))
    a_aug = a_aug.at[:, _NUM_CONCEPTS].set(jnp.bfloat16(1.0))
    d_half = (0.5 * D).astype(jnp.bfloat16)
    return pl.pallas_call(
        _fwd,
        grid=(nb,),
        in_specs=[
            pl.BlockSpec((_NUM_STUDENTS, 2 * _NUM_CONCEPTS), lambda q: (0, 0)),
            pl.BlockSpec((_QB, _NUM_CONCEPTS), lambda q: (q, 0)),
            pl.BlockSpec((1, 1, _QB), lambda q: (q, 0, 0)),
        ],
        out_specs=pl.BlockSpec((_NUM_STUDENTS, _QB), lambda q: (0, q)),
        out_shape=jax.ShapeDtypeStruct((_NUM_STUDENTS, _NUM_QUESTIONS),
                                       jnp.float32),
        compiler_params=pltpu.CompilerParams(
            dimension_semantics=("parallel",)),
    )(a_aug, d_half, c3)


# speedup vs baseline: 1.0437x; 1.0115x over previous
import jax
import jax.numpy as jnp
from jax.experimental import pallas as pl
from jax.experimental.pallas import tpu as pltpu

_NUM_STUDENTS = 4096
_NUM_QUESTIONS = 16384
_NUM_CONCEPTS = 128
_QB = 1024


def _fwd(a_ref, d_ref, c_ref, o_ref):
    c = c_ref[0]
    oh = (c == jax.lax.broadcasted_iota(jnp.int32, (_NUM_CONCEPTS, _QB), 0))
    oh = oh.astype(jnp.bfloat16)
    m = jnp.dot(d_ref[...], oh, preferred_element_type=jnp.float32)
    qi = jax.lax.broadcasted_iota(jnp.int32, (_QB, _QB), 0)
    qj = jax.lax.broadcasted_iota(jnp.int32, (_QB, _QB), 1)
    neg_dh = -jnp.sum(jnp.where(qi == qj, m, 0.0), axis=0, keepdims=True)
    neg_dh = neg_dh.astype(jnp.bfloat16)
    pad = jnp.zeros((_NUM_CONCEPTS - 1, _QB), jnp.bfloat16)
    oh_aug = jnp.concatenate([oh, neg_dh, pad], axis=0)
    z = jnp.dot(a_ref[...], oh_aug, preferred_element_type=jnp.float32)
    o_ref[...] = 0.6 + 0.4 * jnp.tanh(z)


def kernel(x, A, D, concepts_q, concepts_c):
    nb = _NUM_QUESTIONS // _QB
    c3 = concepts_c.reshape(nb, 1, _QB)
    a_aug = jnp.zeros((_NUM_STUDENTS, 2 * _NUM_CONCEPTS), jnp.bfloat16)
    a_aug = a_aug.at[:, :_NUM_CONCEPTS].set((0.5 * A).astype(jnp.bfloat16))
    a_aug = a_aug.at[:, _NUM_CONCEPTS].set(jnp.bfloat16(1.0))
    d_half = (0.5 * D).astype(jnp.bfloat16)
    return pl.pallas_call(
        _fwd,
        grid=(nb,),
        in_specs=[
            pl.BlockSpec((_NUM_STUDENTS, 2 * _NUM_CONCEPTS), lambda q: (0, 0)),
            pl.BlockSpec((_QB, _NUM_CONCEPTS), lambda q: (q, 0)),
            pl.BlockSpec((1, 1, _QB), lambda q: (q, 0, 0)),
        ],
        out_specs=pl.BlockSpec((_NUM_STUDENTS, _QB), lambda q: (0, q)),
        out_shape=jax.ShapeDtypeStruct((_NUM_STUDENTS, _NUM_QUESTIONS),
                                       jnp.float32),
    )(a_aug, d_half, c3)
